# baseline (speedup 1.0000x reference)
import functools

import jax
import jax.numpy as jnp
from jax import lax
from jax.experimental import pallas as pl
from jax.experimental.pallas import tpu as pltpu


def _silu(v):
    h = v * v.dtype.type(0.5)
    t = jnp.tanh(h)
    return h * t + h


def _fused_kernel(x_ref, we_ref, be_ref, wd_ref, bd_ref,
                  ws1_ref, bs1_ref, ws2_ref, bs2_ref,
                  wp_ref, bp_ref, o_ref, *, G, H, W, K):
    HW = H * W
    Cexp = we_ref.shape[1]
    CH = min(256, HW)
    we = we_ref[...]
    be = be_ref[...]
    wd = wd_ref[...]
    bd = bd_ref[...]
    ones_row = jnp.ones((1, HW), jnp.bfloat16)

    ridx = lax.broadcasted_iota(jnp.int32, (CH, Cexp), 0)
    mask_l = (ridx % W != 0).astype(jnp.bfloat16)
    mask_r = (ridx % W != W - 1).astype(jnp.bfloat16)
    zrow = jnp.zeros((1, Cexp), jnp.bfloat16)
    zhalo = jnp.zeros((W, Cexp), jnp.bfloat16)

    acts = []
    pooled = []
    for g in range(G):
        e = lax.dot_general(x_ref[g].astype(jnp.bfloat16), we,
                            (((0,), (0,)), ((), ())),
                            preferred_element_type=jnp.float32)
        eb = _silu(e.astype(jnp.bfloat16) + be)

        chunks = []
        for lo in range(0, HW, CH):
            core = eb[max(lo - W, 0):min(lo + CH + W, HW)]
            parts = ([zhalo] if lo == 0 else []) + [core] + \
                    ([zhalo] if lo + CH == HW else [])
            win = parts[0] if len(parts) == 1 else jnp.concatenate(parts, 0)
            u = [None, None, None]
            for i in range(K):
                sl = win[i * W:i * W + CH]
                for j in range(K):
                    tap = wd[i * K + j:i * K + j + 1, :]
                    t = sl * tap
                    u[j] = t if u[j] is None else u[j] + t
            u0m = u[0] * mask_r
            u2m = u[2] * mask_l
            s = u0m + jnp.concatenate([u2m[2:], zrow, zrow], axis=0)
            a = u[1] + jnp.concatenate([u2m[1:2], s[:-1]], axis=0)
            chunks.append(_silu(a + bd))
        a = jnp.concatenate(chunks, axis=0)
        acts.append(a)
        pooled.append(jnp.dot(ones_row, a, preferred_element_type=jnp.float32))

    p = jnp.concatenate(pooled, axis=0) * (1.0 / float(HW))
    s1 = _silu(jnp.dot(p, ws1_ref[...],
                       preferred_element_type=jnp.float32) + bs1_ref[...])
    uT = lax.dot_general(ws2_ref[...], s1, (((0,), (1,)), ((), ())),
                         preferred_element_type=jnp.float32)
    sT = jax.nn.sigmoid(uT + bs2_ref[...]).astype(jnp.bfloat16)

    wp = wp_ref[...]
    bp = bp_ref[...]
    for g in range(G):
        wg = wp * sT[:, g:g + 1]
        o = lax.dot_general(wg, acts[g], (((0,), (1,)), ((), ())),
                            preferred_element_type=jnp.float32)
        o_ref[g] = (o + bp).reshape(o_ref.shape[1], H, W)


def kernel(x, w_exp, b_exp, w_dw, b_dw, w_se1, b_se1, w_se2, b_se2,
           w_proj, b_proj):
    N, Cin, H, W = x.shape
    Cexp = w_exp.shape[1]
    Cout = w_proj.shape[1]
    Csq = w_se1.shape[1]
    K = w_dw.shape[0]
    HW = H * W
    G = 8 if N % 8 == 0 else 1
    bf16 = jnp.bfloat16

    xb = x.reshape(N, Cin, HW)
    out = pl.pallas_call(
        functools.partial(_fused_kernel, G=G, H=H, W=W, K=K),
        out_shape=jax.ShapeDtypeStruct((N, Cout, H, W), x.dtype),
        grid_spec=pltpu.PrefetchScalarGridSpec(
            num_scalar_prefetch=0, grid=(N // G,),
            in_specs=[
                pl.BlockSpec((G, Cin, HW), lambda n: (n, 0, 0)),
                pl.BlockSpec((Cin, Cexp), lambda n: (0, 0)),
                pl.BlockSpec((1, Cexp), lambda n: (0, 0)),
                pl.BlockSpec((K * K, Cexp), lambda n: (0, 0)),
                pl.BlockSpec((1, Cexp), lambda n: (0, 0)),
                pl.BlockSpec((Cexp, Csq), lambda n: (0, 0)),
                pl.BlockSpec((1, Csq), lambda n: (0, 0)),
                pl.BlockSpec((Csq, Cexp), lambda n: (0, 0)),
                pl.BlockSpec((Cexp, 1), lambda n: (0, 0)),
                pl.BlockSpec((Cexp, Cout), lambda n: (0, 0)),
                pl.BlockSpec((Cout, 1), lambda n: (0, 0)),
            ],
            out_specs=pl.BlockSpec((G, Cout, H, W), lambda n: (n, 0, 0, 0))),
        compiler_params=pltpu.CompilerParams(
            dimension_semantics=("parallel",),
            vmem_limit_bytes=64 * 1024 * 1024),
    )(xb, w_exp.astype(bf16), b_exp.reshape(1, Cexp).astype(bf16),
      w_dw.reshape(K * K, Cexp).astype(bf16),
      b_dw.reshape(1, Cexp).astype(bf16),
      w_se1, b_se1.reshape(1, Csq), w_se2, b_se2.reshape(Cexp, 1),
      w_proj.astype(bf16), b_proj.reshape(Cout, 1))
    return out


# speedup vs baseline: 1.4511x; 1.4511x over previous
import functools

import jax
import jax.numpy as jnp
from jax import lax
from jax.experimental import pallas as pl
from jax.experimental.pallas import tpu as pltpu


def _silu(v):
    h = v * v.dtype.type(0.5)
    t = jnp.tanh(h)
    return h * t + h


def _fused_kernel(x_ref, we_ref, be_ref, wd_ref, bd_ref,
                  ws1_ref, bs1_ref, ws2_ref, bs2_ref,
                  wp_ref, bp_ref, o_ref, *, G, H, W, K):
    HW = H * W
    Cexp = we_ref.shape[1]
    CH = min(256, HW)
    we = we_ref[...]
    be = be_ref[...]
    wd = wd_ref[...]
    bd = bd_ref[...]
    ones_row = jnp.ones((1, HW), jnp.bfloat16)

    ridx = lax.broadcasted_iota(jnp.int32, (CH, Cexp), 0)
    mask_l = (ridx % W != 0).astype(jnp.bfloat16)
    mask_r = (ridx % W != W - 1).astype(jnp.bfloat16)
    zrow = jnp.zeros((1, Cexp), jnp.bfloat16)
    zhalo = jnp.zeros((W, Cexp), jnp.bfloat16)

    acts = []
    pooled = []
    for g in range(G):
        e = lax.dot_general(x_ref[g].astype(jnp.bfloat16), we,
                            (((0,), (0,)), ((), ())),
                            preferred_element_type=jnp.float32)
        eb = _silu(e.astype(jnp.bfloat16) + be)

        chunks = []
        for lo in range(0, HW, CH):
            core = eb[max(lo - W, 0):min(lo + CH + W, HW)]
            parts = ([zhalo] if lo == 0 else []) + [core] + \
                    ([zhalo] if lo + CH == HW else [])
            win = parts[0] if len(parts) == 1 else jnp.concatenate(parts, 0)
            u = [None, None, None]
            for i in range(K):
                sl = win[i * W:i * W + CH]
                for j in range(K):
                    tap = wd[i * K + j:i * K + j + 1, :]
                    t = sl * tap
                    u[j] = t if u[j] is None else u[j] + t
            u0m = u[0] * mask_r
            u2m = u[2] * mask_l
            s = u0m + jnp.concatenate([u2m[2:], zrow, zrow], axis=0)
            a = u[1] + jnp.concatenate([u2m[1:2], s[:-1]], axis=0)
            chunks.append(_silu(a + bd))
        a = jnp.concatenate(chunks, axis=0)
        acts.append(a)
        pooled.append(jnp.dot(ones_row, a, preferred_element_type=jnp.float32))

    p = jnp.concatenate(pooled, axis=0) * (1.0 / float(HW))
    s1 = _silu(jnp.dot(p, ws1_ref[...],
                       preferred_element_type=jnp.float32) + bs1_ref[...])
    uT = lax.dot_general(ws2_ref[...], s1, (((0,), (1,)), ((), ())),
                         preferred_element_type=jnp.float32)
    sT = jax.nn.sigmoid(uT + bs2_ref[...]).astype(jnp.bfloat16)

    wp = wp_ref[...]
    bp = bp_ref[...]
    for g in range(G):
        wg = wp * sT[:, g:g + 1]
        o = lax.dot_general(wg, acts[g], (((0,), (1,)), ((), ())),
                            preferred_element_type=jnp.float32)
        o_ref[g] = o + bp


def kernel(x, w_exp, b_exp, w_dw, b_dw, w_se1, b_se1, w_se2, b_se2,
           w_proj, b_proj):
    N, Cin, H, W = x.shape
    Cexp = w_exp.shape[1]
    Cout = w_proj.shape[1]
    Csq = w_se1.shape[1]
    K = w_dw.shape[0]
    HW = H * W
    G = 16 if N % 16 == 0 else 1
    bf16 = jnp.bfloat16

    xb = x.reshape(N, Cin, HW)
    out = pl.pallas_call(
        functools.partial(_fused_kernel, G=G, H=H, W=W, K=K),
        out_shape=jax.ShapeDtypeStruct((N, Cout, HW), x.dtype),
        grid_spec=pltpu.PrefetchScalarGridSpec(
            num_scalar_prefetch=0, grid=(N // G,),
            in_specs=[
                pl.BlockSpec((G, Cin, HW), lambda n: (n, 0, 0)),
                pl.BlockSpec((Cin, Cexp), lambda n: (0, 0)),
                pl.BlockSpec((1, Cexp), lambda n: (0, 0)),
                pl.BlockSpec((K * K, Cexp), lambda n: (0, 0)),
                pl.BlockSpec((1, Cexp), lambda n: (0, 0)),
                pl.BlockSpec((Cexp, Csq), lambda n: (0, 0)),
                pl.BlockSpec((1, Csq), lambda n: (0, 0)),
                pl.BlockSpec((Csq, Cexp), lambda n: (0, 0)),
                pl.BlockSpec((Cexp, 1), lambda n: (0, 0)),
                pl.BlockSpec((Cexp, Cout), lambda n: (0, 0)),
                pl.BlockSpec((Cout, 1), lambda n: (0, 0)),
            ],
            out_specs=pl.BlockSpec((G, Cout, HW), lambda n: (n, 0, 0))),
        compiler_params=pltpu.CompilerParams(
            dimension_semantics=("parallel",),
            vmem_limit_bytes=64 * 1024 * 1024),
    )(xb, w_exp.astype(bf16), b_exp.reshape(1, Cexp).astype(bf16),
      w_dw.reshape(K * K, Cexp).astype(bf16),
      b_dw.reshape(1, Cexp).astype(bf16),
      w_se1, b_se1.reshape(1, Csq), w_se2, b_se2.reshape(Cexp, 1),
      w_proj.astype(bf16), b_proj.reshape(Cout, 1))
    return out.reshape(N, Cout, H, W)
